# baseline (speedup 1.0000x reference)
import jax
import jax.numpy as jnp
from jax import lax
from jax.experimental import pallas as pl
from jax.experimental.pallas import tpu as pltpu
from jax.experimental.pallas import tpu_sc as plsc

_B = 4096
_NEG = 10
_D = 64
_REG_SCALE = 0.0001
_NE = 1000000
_NR = 1000
_EROWS = 2 * _B + 4 * _B * _NEG
_RROWS = _B + 2 * _B * _NEG

_CB = 8192
_HB = _CB // 2
_G = -(-_NE // _CB)
_PROWS = _G * _HB
_VROWS = 2 * _PROWS
_RCB = 1024
_RHB = _RCB // 2
_RVROWS = _RCB

_NW = 32
_EPW = _EROWS // _NW
_RPW = _RROWS // _NW
_CH = 128
_ECH = _EPW // _CH
_RCH = _RPW // _CH
_UNR_E = 6
_UNR_R = 3


def _repack_body(xt, eye, out):
    del eye
    x = xt[...]
    half = x.shape[1] // 2
    tl = lax.transpose(x[:, :half], (1, 0))
    tr = lax.transpose(x[:, half:], (1, 0))
    out[...] = jnp.concatenate([tl, tr], axis=1)


def _make_repack(cb, grid):
    return pl.pallas_call(
        _repack_body,
        grid=(grid,),
        in_specs=[
            pl.BlockSpec((_D, cb), lambda i: (0, i)),
            pl.BlockSpec((_D, _D), lambda i: (0, 0)),
        ],
        out_specs=pl.BlockSpec((cb // 2, 2 * _D), lambda i: (i, 0)),
        out_shape=jax.ShapeDtypeStruct((grid * (cb // 2), 2 * _D),
                                       jnp.float32),
    )


def _perm(i, cb):
    w = i // cb
    j = i % cb
    return w * cb + 2 * (j % (cb // 2)) + j // (cb // 2)


def _gather_body(ent_hbm, rel_hbm, eidx_hbm, ridx_hbm, eout, rout,
                 eidx_v, ridx_v, b0, b1, b2, b3, b4, b5,
                 g0, g1, g2, g3, g4, g5, w0, w1, w2, w3, w4, w5):
    bufs = (b0, b1, b2, b3, b4, b5)
    gsems = (g0, g1, g2, g3, g4, g5)
    wsems = (w0, w1, w2, w3, w4, w5)
    wid = lax.axis_index("s") * 2 + lax.axis_index("c")
    pltpu.sync_copy(eidx_hbm.at[wid], eidx_v)
    pltpu.sync_copy(ridx_hbm.at[wid], ridx_v)

    def phase(table, idx_v, out, base_row, nbody, unr):
        def body(j, carry):
            gh = []
            for b in range(unr):
                c = j * unr + b
                gh.append(pltpu.async_copy(table.at[idx_v.at[c]],
                                           bufs[b], gsems[b]))
            wh = []
            for b in range(unr):
                c = j * unr + b
                gh[b].wait()
                wh.append(pltpu.async_copy(
                    bufs[b], out.at[pl.ds(base_row + c * _CH, _CH)],
                    wsems[b]))
            for b in range(unr):
                wh[b].wait()
            return carry
        lax.fori_loop(0, nbody, body, 0)

    phase(ent_hbm, eidx_v, eout, wid * _EPW, _ECH // _UNR_E, _UNR_E)
    phase(rel_hbm, ridx_v, rout, wid * _RPW, _RCH // _UNR_R, _UNR_R)


_sc_gather_cache = []


def _get_sc_gather():
    if not _sc_gather_cache:
        _sc_gather_cache.append(pl.kernel(
            _gather_body,
            out_type=(jax.ShapeDtypeStruct((_EROWS, _D), jnp.float32),
                      jax.ShapeDtypeStruct((_RROWS, _D), jnp.float32)),
            mesh=plsc.VectorSubcoreMesh(core_axis_name="c",
                                        subcore_axis_name="s"),
            compiler_params=pltpu.CompilerParams(use_tc_tiling_on_sc=False),
            scratch_types=(
                [pltpu.VMEM((_ECH, _CH), jnp.int32),
                 pltpu.VMEM((_RCH, _CH), jnp.int32)]
                + [pltpu.VMEM((_CH, _D), jnp.float32) for _ in range(6)]
                + [pltpu.SemaphoreType.DMA for _ in range(12)]
            ),
        ))
    return _sc_gather_cache[0]


_C = 1024
_GRID = (_B * _NEG) // (2 * _C)


def _dense_body(mh, mt, mrr, we, wo, a, bb, cc, dd, rh, rt,
                m1d, m2d, ones2, linr, out):
    i = pl.program_id(0)
    m1 = m1d[...]
    m2 = m2d[...]
    o2 = ones2[...]
    lw = linr[0, 0]
    lb = linr[0, 1]
    nt = (((1,), (1,)), ((), ()))

    def psq(x2, y2, rel2):
        ht = jnp.tanh(jnp.dot(x2, m1, preferred_element_type=jnp.float32)
                      + jnp.dot(y2, m2, preferred_element_type=jnp.float32))
        s = rel2 * ht
        q = lax.dot_general(o2, s, nt,
                            preferred_element_type=jnp.float32)
        return jax.nn.sigmoid(q * lw + lb)

    p_hn = psq(a[...], bb[...], rh[...])
    p_tn = psq(cc[...], dd[...], rt[...])
    neg = (jnp.sum(p_hn * p_hn) + jnp.sum(p_tn * p_tn)) * (
        1.0 / (2.0 * _NEG * _B))

    @pl.when(i == 0)
    def _():
        mhv = mh[...]
        mtv = mt[...]
        mrv = mrr[...]
        p = psq(mhv, mtv, mrv)
        dlt_e = p[0:1, :] - we[...]
        dlt_o = p[1:2, :] - wo[...]
        f_h = (jnp.sum(dlt_e * dlt_e) + jnp.sum(dlt_o * dlt_o)) * (1.0 / _B)
        reg = (jnp.sum(mhv * mhv) + jnp.sum(mrv * mrv)
               + jnp.sum(mtv * mtv)) * (_REG_SCALE * 0.5 / _B)
        out[0, 0] = f_h + reg

    out[0, 0] += neg


def _make_dense(interpret=False):
    return pl.pallas_call(
        _dense_body,
        grid=(_GRID,),
        in_specs=[
            pl.BlockSpec((2 * _C, 2 * _D), lambda i: (0, 0)),
            pl.BlockSpec((2 * _C, 2 * _D), lambda i: (1, 0)),
            pl.BlockSpec((2 * _C, 2 * _D), lambda i: (0, 0)),
            pl.BlockSpec((1, 2 * _C), lambda i: (0, 0)),
            pl.BlockSpec((1, 2 * _C), lambda i: (0, 0)),
            pl.BlockSpec((_C, 2 * _D), lambda i: (4 + i, 0)),
            pl.BlockSpec((_C, 2 * _D), lambda i: (24 + i, 0)),
            pl.BlockSpec((_C, 2 * _D), lambda i: (44 + i, 0)),
            pl.BlockSpec((_C, 2 * _D), lambda i: (64 + i, 0)),
            pl.BlockSpec((_C, 2 * _D), lambda i: (2 + i, 0)),
            pl.BlockSpec((_C, 2 * _D), lambda i: (22 + i, 0)),
            pl.BlockSpec((2 * _D, 2 * _D), lambda i: (0, 0)),
            pl.BlockSpec((2 * _D, 2 * _D), lambda i: (0, 0)),
            pl.BlockSpec((2, 2 * _D), lambda i: (0, 0)),
            pl.BlockSpec((1, 2), lambda i: (0, 0)),
        ],
        out_specs=pl.BlockSpec((1, 1), lambda i: (0, 0),
                               memory_space=pltpu.SMEM),
        out_shape=jax.ShapeDtypeStruct((1, 1), jnp.float32),
        interpret=interpret,
    )


_dense = _make_dense()


def _dense_loss(e2, r2, w, mr1, mr2, lin_w, lin_b, dense_fn=None):
    if dense_fn is None:
        dense_fn = _dense
    lin = jnp.concatenate([lin_w.reshape(1, 1), lin_b.reshape(1, 1)], axis=1)
    z = jnp.zeros((_D, _D), jnp.float32)
    m1d = jnp.block([[mr1, z], [z, mr1]])
    m2d = jnp.block([[mr2, z], [z, mr2]])
    o = jnp.ones((1, _D), jnp.float32)
    zo = jnp.zeros((1, _D), jnp.float32)
    ones2 = jnp.concatenate(
        [jnp.concatenate([o, zo], axis=1),
         jnp.concatenate([zo, o], axis=1)], axis=0)
    we = w[0::2].reshape(1, _B // 2)
    wo = w[1::2].reshape(1, _B // 2)
    out = dense_fn(e2, e2, r2, we, wo, e2, e2, e2, e2, r2, r2,
                   m1d, m2d, ones2, lin)
    return out[0, 0]


def kernel(h, r, t, w, n_hn, n_rel_hn, n_t, n_h, n_rel_tn, n_tn,
           s_h, s_r, s_t, s_w, ent_emb, rel_emb, mr1, mr2, lin_w, lin_b):
    eye = jnp.eye(_D, dtype=jnp.float32)
    epacked = _make_repack(_CB, _G)(ent_emb.T, eye)
    rpacked = _make_repack(_RCB, 1)(rel_emb.T, eye)
    etab = epacked.reshape(_VROWS, _D)
    rtab = rpacked.reshape(_RVROWS, _D)
    ei = _perm(jnp.concatenate([
        h, t, n_hn.reshape(-1), n_t.reshape(-1),
        n_h.reshape(-1), n_tn.reshape(-1)]).astype(jnp.int32), _CB)
    ri = _perm(jnp.concatenate([
        r, n_rel_hn.reshape(-1), n_rel_tn.reshape(-1)]).astype(jnp.int32),
        _RCB)
    erows, rrows = _get_sc_gather()(etab, rtab,
                                    ei.reshape(_NW, _ECH, _CH),
                                    ri.reshape(_NW, _RCH, _CH))
    e2 = erows.reshape(_EROWS // 2, 2 * _D)
    r2 = rrows.reshape(_RROWS // 2, 2 * _D)
    return _dense_loss(e2, r2, w, mr1, mr2, lin_w, lin_b)


# speedup vs baseline: 1.0902x; 1.0902x over previous
import jax
import jax.numpy as jnp
from jax import lax
from jax.experimental import pallas as pl
from jax.experimental.pallas import tpu as pltpu
from jax.experimental.pallas import tpu_sc as plsc

_B = 4096
_NEG = 10
_D = 64
_REG_SCALE = 0.0001
_NE = 1000000
_NR = 1000
_EROWS = 2 * _B + 4 * _B * _NEG
_RROWS = _B + 2 * _B * _NEG

_CB = 16384
_HB = _CB // 2
_G = -(-_NE // _CB)
_PROWS = _G * _HB
_VROWS = 2 * _PROWS
_RCB = 1024
_RHB = _RCB // 2
_RVROWS = _RCB

_NW = 32
_EPW = _EROWS // _NW
_RPW = _RROWS // _NW
_CH = 128
_ECH = _EPW // _CH
_RCH = _RPW // _CH
_UNR_E = 6
_UNR_R = 3


def _repack_body(xt, eye, out):
    del eye
    x = xt[...]
    half = x.shape[1] // 2
    tl = lax.transpose(x[:, :half], (1, 0))
    tr = lax.transpose(x[:, half:], (1, 0))
    out[...] = jnp.concatenate([tl, tr], axis=1)


def _make_repack(cb, grid):
    return pl.pallas_call(
        _repack_body,
        grid=(grid,),
        in_specs=[
            pl.BlockSpec((_D, cb), lambda i: (0, i)),
            pl.BlockSpec((_D, _D), lambda i: (0, 0)),
        ],
        out_specs=pl.BlockSpec((cb // 2, 2 * _D), lambda i: (i, 0)),
        out_shape=jax.ShapeDtypeStruct((grid * (cb // 2), 2 * _D),
                                       jnp.float32),
    )


def _perm(i, cb):
    w = i // cb
    j = i % cb
    return w * cb + 2 * (j % (cb // 2)) + j // (cb // 2)


def _gather_body(ent_hbm, rel_hbm, eidx_hbm, ridx_hbm, eout, rout,
                 eidx_v, ridx_v, b0, b1, b2, b3, b4, b5,
                 g0, g1, g2, g3, g4, g5, w0, w1, w2, w3, w4, w5):
    bufs = (b0, b1, b2, b3, b4, b5)
    gsems = (g0, g1, g2, g3, g4, g5)
    wsems = (w0, w1, w2, w3, w4, w5)
    wid = lax.axis_index("s") * 2 + lax.axis_index("c")
    pltpu.sync_copy(eidx_hbm.at[wid], eidx_v)
    pltpu.sync_copy(ridx_hbm.at[wid], ridx_v)

    def phase(table, idx_v, out, base_row, nbody, unr):
        def body(j, carry):
            gh = []
            for b in range(unr):
                c = j * unr + b
                gh.append(pltpu.async_copy(table.at[idx_v.at[c]],
                                           bufs[b], gsems[b]))
            wh = []
            for b in range(unr):
                c = j * unr + b
                gh[b].wait()
                wh.append(pltpu.async_copy(
                    bufs[b], out.at[pl.ds(base_row + c * _CH, _CH)],
                    wsems[b]))
            for b in range(unr):
                wh[b].wait()
            return carry
        lax.fori_loop(0, nbody, body, 0)

    phase(ent_hbm, eidx_v, eout, wid * _EPW, _ECH // _UNR_E, _UNR_E)
    phase(rel_hbm, ridx_v, rout, wid * _RPW, _RCH // _UNR_R, _UNR_R)


_sc_gather_cache = []


def _get_sc_gather():
    if not _sc_gather_cache:
        _sc_gather_cache.append(pl.kernel(
            _gather_body,
            out_type=(jax.ShapeDtypeStruct((_EROWS, _D), jnp.float32),
                      jax.ShapeDtypeStruct((_RROWS, _D), jnp.float32)),
            mesh=plsc.VectorSubcoreMesh(core_axis_name="c",
                                        subcore_axis_name="s"),
            compiler_params=pltpu.CompilerParams(use_tc_tiling_on_sc=False),
            scratch_types=(
                [pltpu.VMEM((_ECH, _CH), jnp.int32),
                 pltpu.VMEM((_RCH, _CH), jnp.int32)]
                + [pltpu.VMEM((_CH, _D), jnp.float32) for _ in range(6)]
                + [pltpu.SemaphoreType.DMA for _ in range(12)]
            ),
        ))
    return _sc_gather_cache[0]


_C = 1024
_GRID = (_B * _NEG) // (2 * _C)


def _dense_body(mh, mt, mrr, we, wo, a, bb, cc, dd, rh, rt,
                m1d, m2d, ones2, linr, out):
    i = pl.program_id(0)
    m1 = m1d[...]
    m2 = m2d[...]
    o2 = ones2[...]
    lw = linr[0, 0]
    lb = linr[0, 1]
    nt = (((1,), (1,)), ((), ()))

    def psq(x2, y2, rel2):
        ht = jnp.tanh(jnp.dot(x2, m1, preferred_element_type=jnp.float32)
                      + jnp.dot(y2, m2, preferred_element_type=jnp.float32))
        s = rel2 * ht
        q = lax.dot_general(o2, s, nt,
                            preferred_element_type=jnp.float32)
        return jax.nn.sigmoid(q * lw + lb)

    p_hn = psq(a[...], bb[...], rh[...])
    p_tn = psq(cc[...], dd[...], rt[...])
    neg = (jnp.sum(p_hn * p_hn) + jnp.sum(p_tn * p_tn)) * (
        1.0 / (2.0 * _NEG * _B))

    @pl.when(i == 0)
    def _():
        mhv = mh[...]
        mtv = mt[...]
        mrv = mrr[...]
        p = psq(mhv, mtv, mrv)
        dlt_e = p[0:1, :] - we[...]
        dlt_o = p[1:2, :] - wo[...]
        f_h = (jnp.sum(dlt_e * dlt_e) + jnp.sum(dlt_o * dlt_o)) * (1.0 / _B)
        reg = (jnp.sum(mhv * mhv) + jnp.sum(mrv * mrv)
               + jnp.sum(mtv * mtv)) * (_REG_SCALE * 0.5 / _B)
        out[0, 0] = f_h + reg

    out[0, 0] += neg


def _make_dense(interpret=False):
    return pl.pallas_call(
        _dense_body,
        grid=(_GRID,),
        in_specs=[
            pl.BlockSpec((2 * _C, 2 * _D), lambda i: (0, 0)),
            pl.BlockSpec((2 * _C, 2 * _D), lambda i: (1, 0)),
            pl.BlockSpec((2 * _C, 2 * _D), lambda i: (0, 0)),
            pl.BlockSpec((1, 2 * _C), lambda i: (0, 0)),
            pl.BlockSpec((1, 2 * _C), lambda i: (0, 0)),
            pl.BlockSpec((_C, 2 * _D), lambda i: (4 + i, 0)),
            pl.BlockSpec((_C, 2 * _D), lambda i: (24 + i, 0)),
            pl.BlockSpec((_C, 2 * _D), lambda i: (44 + i, 0)),
            pl.BlockSpec((_C, 2 * _D), lambda i: (64 + i, 0)),
            pl.BlockSpec((_C, 2 * _D), lambda i: (2 + i, 0)),
            pl.BlockSpec((_C, 2 * _D), lambda i: (22 + i, 0)),
            pl.BlockSpec((2 * _D, 2 * _D), lambda i: (0, 0)),
            pl.BlockSpec((2 * _D, 2 * _D), lambda i: (0, 0)),
            pl.BlockSpec((2, 2 * _D), lambda i: (0, 0)),
            pl.BlockSpec((1, 2), lambda i: (0, 0)),
        ],
        out_specs=pl.BlockSpec((1, 1), lambda i: (0, 0),
                               memory_space=pltpu.SMEM),
        out_shape=jax.ShapeDtypeStruct((1, 1), jnp.float32),
        interpret=interpret,
    )


_dense = _make_dense()


def _dense_loss(e2, r2, w, mr1, mr2, lin_w, lin_b, dense_fn=None):
    if dense_fn is None:
        dense_fn = _dense
    lin = jnp.concatenate([lin_w.reshape(1, 1), lin_b.reshape(1, 1)], axis=1)
    z = jnp.zeros((_D, _D), jnp.float32)
    m1d = jnp.block([[mr1, z], [z, mr1]])
    m2d = jnp.block([[mr2, z], [z, mr2]])
    o = jnp.ones((1, _D), jnp.float32)
    zo = jnp.zeros((1, _D), jnp.float32)
    ones2 = jnp.concatenate(
        [jnp.concatenate([o, zo], axis=1),
         jnp.concatenate([zo, o], axis=1)], axis=0)
    we = w[0::2].reshape(1, _B // 2)
    wo = w[1::2].reshape(1, _B // 2)
    out = dense_fn(e2, e2, r2, we, wo, e2, e2, e2, e2, r2, r2,
                   m1d, m2d, ones2, lin)
    return out[0, 0]


def kernel(h, r, t, w, n_hn, n_rel_hn, n_t, n_h, n_rel_tn, n_tn,
           s_h, s_r, s_t, s_w, ent_emb, rel_emb, mr1, mr2, lin_w, lin_b):
    eye = jnp.eye(_D, dtype=jnp.float32)
    epacked = _make_repack(_CB, _G)(ent_emb.T, eye)
    rpacked = _make_repack(_RCB, 1)(rel_emb.T, eye)
    etab = epacked.reshape(_VROWS, _D)
    rtab = rpacked.reshape(_RVROWS, _D)
    ei = _perm(jnp.concatenate([
        h, t, n_hn.reshape(-1), n_t.reshape(-1),
        n_h.reshape(-1), n_tn.reshape(-1)]).astype(jnp.int32), _CB)
    ri = _perm(jnp.concatenate([
        r, n_rel_hn.reshape(-1), n_rel_tn.reshape(-1)]).astype(jnp.int32),
        _RCB)
    erows, rrows = _get_sc_gather()(etab, rtab,
                                    ei.reshape(_NW, _ECH, _CH),
                                    ri.reshape(_NW, _RCH, _CH))
    e2 = erows.reshape(_EROWS // 2, 2 * _D)
    r2 = rrows.reshape(_RROWS // 2, 2 * _D)
    return _dense_loss(e2, r2, w, mr1, mr2, lin_w, lin_b)


# speedup vs baseline: 1.1356x; 1.0417x over previous
import jax
import jax.numpy as jnp
from jax import lax
from jax.experimental import pallas as pl
from jax.experimental.pallas import tpu as pltpu
from jax.experimental.pallas import tpu_sc as plsc

_B = 4096
_NEG = 10
_D = 64
_REG_SCALE = 0.0001
_NE = 1000000
_NR = 1000
_EROWS = 2 * _B + 4 * _B * _NEG
_RROWS = _B + 2 * _B * _NEG

_CB = 32768
_HB = _CB // 2
_G = -(-_NE // _CB)
_PROWS = _G * _HB
_VROWS = 2 * _PROWS
_RCB = 1024
_RHB = _RCB // 2
_RVROWS = _RCB

_NW = 32
_EPW = _EROWS // _NW
_RPW = _RROWS // _NW
_CH = 128
_ECH = _EPW // _CH
_RCH = _RPW // _CH
_UNR_E = 6
_UNR_R = 3


def _repack_body(xt, eye, out):
    del eye
    x = xt[...]
    half = x.shape[1] // 2
    tl = lax.transpose(x[:, :half], (1, 0))
    tr = lax.transpose(x[:, half:], (1, 0))
    out[...] = jnp.concatenate([tl, tr], axis=1)


def _make_repack(cb, grid):
    return pl.pallas_call(
        _repack_body,
        grid=(grid,),
        in_specs=[
            pl.BlockSpec((_D, cb), lambda i: (0, i)),
            pl.BlockSpec((_D, _D), lambda i: (0, 0)),
        ],
        out_specs=pl.BlockSpec((cb // 2, 2 * _D), lambda i: (i, 0)),
        out_shape=jax.ShapeDtypeStruct((grid * (cb // 2), 2 * _D),
                                       jnp.float32),
    )


def _perm(i, cb):
    w = i // cb
    j = i % cb
    return w * cb + 2 * (j % (cb // 2)) + j // (cb // 2)


def _gather_body(ent_hbm, rel_hbm, eidx_hbm, ridx_hbm, eout, rout,
                 eidx_v, ridx_v, b0, b1, b2, b3, b4, b5,
                 g0, g1, g2, g3, g4, g5, w0, w1, w2, w3, w4, w5):
    bufs = (b0, b1, b2, b3, b4, b5)
    gsems = (g0, g1, g2, g3, g4, g5)
    wsems = (w0, w1, w2, w3, w4, w5)
    wid = lax.axis_index("s") * 2 + lax.axis_index("c")
    pltpu.sync_copy(eidx_hbm.at[wid], eidx_v)
    pltpu.sync_copy(ridx_hbm.at[wid], ridx_v)

    def phase(table, idx_v, out, base_row, nbody, unr):
        def body(j, carry):
            gh = []
            for b in range(unr):
                c = j * unr + b
                gh.append(pltpu.async_copy(table.at[idx_v.at[c]],
                                           bufs[b], gsems[b]))
            wh = []
            for b in range(unr):
                c = j * unr + b
                gh[b].wait()
                wh.append(pltpu.async_copy(
                    bufs[b], out.at[pl.ds(base_row + c * _CH, _CH)],
                    wsems[b]))
            for b in range(unr):
                wh[b].wait()
            return carry
        lax.fori_loop(0, nbody, body, 0)

    phase(ent_hbm, eidx_v, eout, wid * _EPW, _ECH // _UNR_E, _UNR_E)
    phase(rel_hbm, ridx_v, rout, wid * _RPW, _RCH // _UNR_R, _UNR_R)


_sc_gather_cache = []


def _get_sc_gather():
    if not _sc_gather_cache:
        _sc_gather_cache.append(pl.kernel(
            _gather_body,
            out_type=(jax.ShapeDtypeStruct((_EROWS, _D), jnp.float32),
                      jax.ShapeDtypeStruct((_RROWS, _D), jnp.float32)),
            mesh=plsc.VectorSubcoreMesh(core_axis_name="c",
                                        subcore_axis_name="s"),
            compiler_params=pltpu.CompilerParams(use_tc_tiling_on_sc=False),
            scratch_types=(
                [pltpu.VMEM((_ECH, _CH), jnp.int32),
                 pltpu.VMEM((_RCH, _CH), jnp.int32)]
                + [pltpu.VMEM((_CH, _D), jnp.float32) for _ in range(6)]
                + [pltpu.SemaphoreType.DMA for _ in range(12)]
            ),
        ))
    return _sc_gather_cache[0]


_C = 1024
_GRID = (_B * _NEG) // (2 * _C)


def _dense_body(mh, mt, mrr, we, wo, a, bb, cc, dd, rh, rt,
                m1d, m2d, ones2, linr, out):
    i = pl.program_id(0)
    m1 = m1d[...]
    m2 = m2d[...]
    o2 = ones2[...]
    lw = linr[0, 0]
    lb = linr[0, 1]
    nt = (((1,), (1,)), ((), ()))

    def psq(x2, y2, rel2):
        ht = jnp.tanh(jnp.dot(x2, m1, preferred_element_type=jnp.float32)
                      + jnp.dot(y2, m2, preferred_element_type=jnp.float32))
        s = rel2 * ht
        q = lax.dot_general(o2, s, nt,
                            preferred_element_type=jnp.float32)
        return jax.nn.sigmoid(q * lw + lb)

    p_hn = psq(a[...], bb[...], rh[...])
    p_tn = psq(cc[...], dd[...], rt[...])
    neg = (jnp.sum(p_hn * p_hn) + jnp.sum(p_tn * p_tn)) * (
        1.0 / (2.0 * _NEG * _B))

    @pl.when(i == 0)
    def _():
        mhv = mh[...]
        mtv = mt[...]
        mrv = mrr[...]
        p = psq(mhv, mtv, mrv)
        dlt_e = p[0:1, :] - we[...]
        dlt_o = p[1:2, :] - wo[...]
        f_h = (jnp.sum(dlt_e * dlt_e) + jnp.sum(dlt_o * dlt_o)) * (1.0 / _B)
        reg = (jnp.sum(mhv * mhv) + jnp.sum(mrv * mrv)
               + jnp.sum(mtv * mtv)) * (_REG_SCALE * 0.5 / _B)
        out[0, 0] = f_h + reg

    out[0, 0] += neg


def _make_dense(interpret=False):
    return pl.pallas_call(
        _dense_body,
        grid=(_GRID,),
        in_specs=[
            pl.BlockSpec((2 * _C, 2 * _D), lambda i: (0, 0)),
            pl.BlockSpec((2 * _C, 2 * _D), lambda i: (1, 0)),
            pl.BlockSpec((2 * _C, 2 * _D), lambda i: (0, 0)),
            pl.BlockSpec((1, 2 * _C), lambda i: (0, 0)),
            pl.BlockSpec((1, 2 * _C), lambda i: (0, 0)),
            pl.BlockSpec((_C, 2 * _D), lambda i: (4 + i, 0)),
            pl.BlockSpec((_C, 2 * _D), lambda i: (24 + i, 0)),
            pl.BlockSpec((_C, 2 * _D), lambda i: (44 + i, 0)),
            pl.BlockSpec((_C, 2 * _D), lambda i: (64 + i, 0)),
            pl.BlockSpec((_C, 2 * _D), lambda i: (2 + i, 0)),
            pl.BlockSpec((_C, 2 * _D), lambda i: (22 + i, 0)),
            pl.BlockSpec((2 * _D, 2 * _D), lambda i: (0, 0)),
            pl.BlockSpec((2 * _D, 2 * _D), lambda i: (0, 0)),
            pl.BlockSpec((2, 2 * _D), lambda i: (0, 0)),
            pl.BlockSpec((1, 2), lambda i: (0, 0)),
        ],
        out_specs=pl.BlockSpec((1, 1), lambda i: (0, 0),
                               memory_space=pltpu.SMEM),
        out_shape=jax.ShapeDtypeStruct((1, 1), jnp.float32),
        interpret=interpret,
    )


_dense = _make_dense()


def _dense_loss(e2, r2, w, mr1, mr2, lin_w, lin_b, dense_fn=None):
    if dense_fn is None:
        dense_fn = _dense
    lin = jnp.concatenate([lin_w.reshape(1, 1), lin_b.reshape(1, 1)], axis=1)
    z = jnp.zeros((_D, _D), jnp.float32)
    m1d = jnp.block([[mr1, z], [z, mr1]])
    m2d = jnp.block([[mr2, z], [z, mr2]])
    o = jnp.ones((1, _D), jnp.float32)
    zo = jnp.zeros((1, _D), jnp.float32)
    ones2 = jnp.concatenate(
        [jnp.concatenate([o, zo], axis=1),
         jnp.concatenate([zo, o], axis=1)], axis=0)
    we = w[0::2].reshape(1, _B // 2)
    wo = w[1::2].reshape(1, _B // 2)
    out = dense_fn(e2, e2, r2, we, wo, e2, e2, e2, e2, r2, r2,
                   m1d, m2d, ones2, lin)
    return out[0, 0]


def kernel(h, r, t, w, n_hn, n_rel_hn, n_t, n_h, n_rel_tn, n_tn,
           s_h, s_r, s_t, s_w, ent_emb, rel_emb, mr1, mr2, lin_w, lin_b):
    eye = jnp.eye(_D, dtype=jnp.float32)
    epacked = _make_repack(_CB, _G)(ent_emb.T, eye)
    rpacked = _make_repack(_RCB, 1)(rel_emb.T, eye)
    etab = epacked.reshape(_VROWS, _D)
    rtab = rpacked.reshape(_RVROWS, _D)
    ei = _perm(jnp.concatenate([
        h, t, n_hn.reshape(-1), n_t.reshape(-1),
        n_h.reshape(-1), n_tn.reshape(-1)]).astype(jnp.int32), _CB)
    ri = _perm(jnp.concatenate([
        r, n_rel_hn.reshape(-1), n_rel_tn.reshape(-1)]).astype(jnp.int32),
        _RCB)
    erows, rrows = _get_sc_gather()(etab, rtab,
                                    ei.reshape(_NW, _ECH, _CH),
                                    ri.reshape(_NW, _RCH, _CH))
    e2 = erows.reshape(_EROWS // 2, 2 * _D)
    r2 = rrows.reshape(_RROWS // 2, 2 * _D)
    return _dense_loss(e2, r2, w, mr1, mr2, lin_w, lin_b)
